# baseline (speedup 1.0000x reference)
import jax
import jax.numpy as jnp
from jax.experimental import pallas as pl
from jax.experimental.pallas import tpu as pltpu


def _gcn_kernel(feat_ref, wt_ref, adj_ref, out_ref):
    t = jnp.dot(adj_ref[:].astype(jnp.bfloat16), feat_ref[:].astype(jnp.bfloat16),
                preferred_element_type=jnp.float32)
    acc = jnp.dot(t, wt_ref[:], preferred_element_type=jnp.float32)
    out_ref[:] = jnp.maximum(acc, 0.0)


def kernel(feat, adj, W):
    n, in_ft = feat.shape
    out_ft = W.shape[0]

    bm = 400
    out = pl.pallas_call(
        _gcn_kernel,
        grid=(n // bm,),
        in_specs=[
            pl.BlockSpec((n, in_ft), lambda i: (0, 0)),
            pl.BlockSpec((in_ft, out_ft), lambda i: (0, 0)),
            pl.BlockSpec((bm, n), lambda i: (i, 0)),
        ],
        out_specs=pl.BlockSpec((bm, out_ft), lambda i: (i, 0)),
        out_shape=jax.ShapeDtypeStruct((n, out_ft), jnp.float32),
        compiler_params=pltpu.CompilerParams(
            dimension_semantics=("arbitrary",),
        ),
    )(feat, W.T, adj)
    return out
